# baseline (speedup 1.0000x reference)
import functools

import jax
import jax.numpy as jnp
from jax import lax
from jax.experimental import pallas as pl
from jax.experimental.pallas import tpu as pltpu
from jax.experimental.pallas import tpu_sc as plsc

_VOCAB = 100000
_EMBED = 64
_NCTX = 20
_FAN1 = _NCTX * _EMBED
_HIDDEN = 256
_BK = 16


def _sc_gather(x, emb):
    mesh = plsc.VectorSubcoreMesh(core_axis_name="c", subcore_axis_name="s")

    @functools.partial(
        pl.kernel,
        mesh=mesh,
        compiler_params=pltpu.CompilerParams(use_tc_tiling_on_sc=False),
        out_type=jax.ShapeDtypeStruct((_NCTX, _EMBED), jnp.float32),
        scratch_types=[
            pltpu.VMEM((_NCTX,), jnp.int32),
            pltpu.VMEM((_NCTX, _EMBED), jnp.float32),
            pltpu.SemaphoreType.DMA,
        ],
    )
    def gather_kernel(idx_hbm, table_hbm, out_hbm, idx_v, rows_v, sem):
        cid = lax.axis_index("c")
        sid = lax.axis_index("s")

        @pl.when(jnp.logical_and(cid == 0, sid == 0))
        def _():
            pltpu.sync_copy(idx_hbm, idx_v)
            pltpu.async_copy(table_hbm.at[idx_v], rows_v, sem).wait()
            pltpu.sync_copy(rows_v, out_hbm)

    return gather_kernel(x, emb)


_NSTREAM = 4
_NSTEP = _HIDDEN // (_BK * _NSTREAM)


def _tc_body(e_ref, w1_ref, b1_ref, w2a_ref, w2b_ref, w2c_ref, w2d_ref,
             b2_ref, out_ref, h_ref):
    i = pl.program_id(0)

    @pl.when(i == 0)
    def _():
        h = jnp.dot(e_ref[...], w1_ref[...],
                    preferred_element_type=jnp.float32) + b1_ref[...]
        h_ref[...] = jnp.maximum(h, 0.0)

    for k in range(_NSTEP):
        @pl.when(i == k)
        def _(k=k):
            part = (w2a_ref[0:1, :] + w2b_ref[0:1, :]
                    + w2c_ref[0:1, :] + w2d_ref[0:1, :])
            if k == 0:
                out_ref[...] = part + b2_ref[...]
            else:
                out_ref[...] += part

    @pl.when(i == _NSTEP - 1)
    def _():
        full = out_ref[...]
        m = jnp.max(full)
        s = jnp.sum(jnp.exp(full - m))
        out_ref[...] = full - (m + jnp.log(s))


def _tc_dense(e2d, W1, b1_2d, W2, b2_2d):
    w2_specs = [
        pl.BlockSpec((_BK, _VOCAB), (lambda i, j=j: (_NSTREAM * i + j, 0)))
        for j in range(_NSTREAM)
    ]
    return pl.pallas_call(
        _tc_body,
        grid=(_NSTEP,),
        in_specs=[
            pl.BlockSpec((1, _FAN1), lambda i: (0, 0)),
            pl.BlockSpec((_FAN1, _HIDDEN), lambda i: (0, 0)),
            pl.BlockSpec((1, _HIDDEN), lambda i: (0, 0)),
            *w2_specs,
            pl.BlockSpec((1, _VOCAB), lambda i: (0, 0)),
        ],
        out_specs=pl.BlockSpec((1, _VOCAB), lambda i: (0, 0)),
        out_shape=jax.ShapeDtypeStruct((1, _VOCAB), jnp.float32),
        scratch_shapes=[pltpu.VMEM((1, _HIDDEN), jnp.float32)],
    )(e2d, W1, b1_2d, W2, W2, W2, W2, b2_2d)


def kernel(x, emb, W1, b1, W2, b2):
    e = jnp.take(emb, x, axis=0)
    out = _tc_dense(e.reshape(1, _FAN1), W1, b1.reshape(1, _HIDDEN),
                    W2, b2.reshape(1, _VOCAB))
    return out


# speedup vs baseline: 1.8383x; 1.8383x over previous
import functools

import jax
import jax.numpy as jnp
from jax import lax
from jax.experimental import pallas as pl
from jax.experimental.pallas import tpu as pltpu
from jax.experimental.pallas import tpu_sc as plsc

_VOCAB = 100000
_EMBED = 64
_NCTX = 20
_FAN1 = _NCTX * _EMBED
_HIDDEN = 256
_BV = 8192
_NB = -(-_VOCAB // _BV)
_OUTW = _NB * _BV
_NEG = -1e30


def _tc_body(e_ref, w1_ref, b1_ref, w2t_ref, b2_ref, out_ref, h_ref):
    i = pl.program_id(0)

    @pl.when(i == 0)
    def _():
        h = jnp.dot(e_ref[...], w1_ref[...],
                    preferred_element_type=jnp.float32) + b1_ref[...]
        h_ref[...] = jnp.maximum(h, 0.0)

    part = lax.dot_general(h_ref[...], w2t_ref[...],
                           (((1,), (1,)), ((), ())),
                           preferred_element_type=jnp.float32)
    col = i * _BV + lax.broadcasted_iota(jnp.int32, (1, _BV), 1)
    logits = jnp.where(col < _VOCAB, part + b2_ref[...], _NEG)
    out_ref[:, pl.ds(i * _BV, _BV)] = logits

    @pl.when(i == _NB - 1)
    def _():
        full = out_ref[...]
        m = jnp.max(full)
        s = jnp.sum(jnp.exp(full - m))
        out_ref[...] = full - (m + jnp.log(s))


def _tc_dense(e2d, W1, b1_2d, W2t, b2_2d):
    return pl.pallas_call(
        _tc_body,
        grid=(_NB,),
        in_specs=[
            pl.BlockSpec((1, _FAN1), lambda i: (0, 0)),
            pl.BlockSpec((_FAN1, _HIDDEN), lambda i: (0, 0)),
            pl.BlockSpec((1, _HIDDEN), lambda i: (0, 0)),
            pl.BlockSpec((_BV, _HIDDEN), lambda i: (i, 0)),
            pl.BlockSpec((1, _BV), lambda i: (0, i)),
        ],
        out_specs=pl.BlockSpec((1, _OUTW), lambda i: (0, 0)),
        out_shape=jax.ShapeDtypeStruct((1, _OUTW), jnp.float32),
        scratch_shapes=[pltpu.VMEM((1, _HIDDEN), jnp.float32)],
    )(e2d, W1, b1_2d, W2t, b2_2d)


def kernel(x, emb, W1, b1, W2, b2):
    e = jnp.take(emb, x, axis=0)
    out = _tc_dense(e.reshape(1, _FAN1), W1, b1.reshape(1, _HIDDEN),
                    W2.T, b2.reshape(1, _VOCAB))
    return out[:, :_VOCAB]


# speedup vs baseline: 4.1484x; 2.2567x over previous
import jax
import jax.numpy as jnp
from jax import lax
from jax.experimental import pallas as pl
from jax.experimental.pallas import tpu as pltpu

_VOCAB = 100000
_EMBED = 64
_NCTX = 20
_HIDDEN = 256
_BV = 8192
_NB = -(-_VOCAB // _BV)
_OUTW = _NB * _BV
_NEG = -1e30


def _body(x_ref, *refs):
    win_refs = refs[:_NCTX]
    w1_ref, b1_ref, w2t_ref, b2_ref, out_ref, h_ref = refs[_NCTX:]
    i = pl.program_id(0)

    @pl.when(i == 0)
    def _():
        h = b1_ref[...]
        for k in range(_NCTX):
            lane = x_ref[k] % 128
            m = lax.broadcasted_iota(jnp.int32, (_EMBED, 128), 1) == lane
            col = jnp.sum(jnp.where(m, win_refs[k][...], 0.0), axis=1,
                          keepdims=True)
            h = h + lax.dot_general(
                col, w1_ref[k * _EMBED:(k + 1) * _EMBED, :],
                (((0,), (0,)), ((), ())),
                preferred_element_type=jnp.float32)
        h_ref[...] = jnp.maximum(h, 0.0)

    part = lax.dot_general(h_ref[...], w2t_ref[...],
                           (((1,), (1,)), ((), ())),
                           preferred_element_type=jnp.float32)
    col = i * _BV + lax.broadcasted_iota(jnp.int32, (1, _BV), 1)
    logits = jnp.where(col < _VOCAB, part + b2_ref[...], _NEG)
    out_ref[:, pl.ds(i * _BV, _BV)] = logits

    @pl.when(i == _NB - 1)
    def _():
        full = out_ref[...]
        m = jnp.max(full)
        s = jnp.sum(jnp.exp(full - m))
        out_ref[...] = full - (m + jnp.log(s))


_WIN_SPECS = [
    pl.BlockSpec((_EMBED, 128), (lambda i, xr, k=k: (0, xr[k] // 128)))
    for k in range(_NCTX)
]

_GRID_SPEC = pltpu.PrefetchScalarGridSpec(
    num_scalar_prefetch=1,
    grid=(_NB,),
    in_specs=[
        *_WIN_SPECS,
        pl.BlockSpec((_NCTX * _EMBED, _HIDDEN), lambda i, xr: (0, 0)),
        pl.BlockSpec((1, _HIDDEN), lambda i, xr: (0, 0)),
        pl.BlockSpec((_BV, _HIDDEN), lambda i, xr: (i, 0)),
        pl.BlockSpec((1, _BV), lambda i, xr: (0, i)),
    ],
    out_specs=pl.BlockSpec((1, _OUTW), lambda i, xr: (0, 0)),
    scratch_shapes=[pltpu.VMEM((1, _HIDDEN), jnp.float32)],
)


def kernel(x, emb, W1, b1, W2, b2):
    embT = emb.T
    args = [embT] * _NCTX + [W1, b1.reshape(1, _HIDDEN), W2.T,
                             b2.reshape(1, _VOCAB)]
    out = pl.pallas_call(
        _body,
        grid_spec=_GRID_SPEC,
        out_shape=jax.ShapeDtypeStruct((1, _OUTW), jnp.float32),
    )(x.astype(jnp.int32), *args)
    return out[:, :_VOCAB]
